# baseline (speedup 1.0000x reference)
import jax
import jax.numpy as jnp
from jax.experimental import pallas as pl
from jax.experimental.pallas import tpu as pltpu

_GRID = 10
_G2 = _GRID * _GRID
_D = 128
_CH = 4096
_B = 16384
_N = _B // _CH


def _nt(a, b):
    return jax.lax.dot_general(a, b, (((1,), (1,)), ((), ())),
                               preferred_element_type=jnp.float32)


def _tt(w, act):
    return jax.lax.dot_general(w, act, (((0,), (0,)), ((), ())),
                               preferred_element_type=jnp.float32)


def _tt_bf(w, act):
    return jax.lax.dot_general(w.astype(jnp.bfloat16), act.astype(jnp.bfloat16),
                               (((0,), (0,)), ((), ())),
                               preferred_element_type=jnp.float32)


def _pipeline(x_hbm, *rest):
    (flat_hbm, We1_hbm, be1_hbm, We2_hbm, be2_hbm, We3_hbm, be3_hbm,
     Wd1_hbm, bd1_hbm, Wd2_hbm, bd2_hbm, Wd3_hbm, bd3_hbm,
     Wg1_hbm, bg1_hbm, Wg2_hbm, bg2_hbm,
     out_hbm, x_vmem, o_vmem,
     flat_ref, We1_ref, be1_ref, We2_ref, be2_ref, We3_ref, be3_ref,
     Wd1_ref, bd1_ref, Wd2_ref, bd2_ref, Wd3_ref, bd3_ref,
     Wg1_ref, bg1_ref, Wg2_ref, bg2_ref,
     in_sem, out_sem, w_sem) = rest
    eps = 1e-12

    hbm_ws = (flat_hbm, We1_hbm, be1_hbm, We2_hbm, be2_hbm, We3_hbm,
              be3_hbm, Wd1_hbm, bd1_hbm, Wd2_hbm, bd2_hbm, Wd3_hbm,
              bd3_hbm, Wg1_hbm, bg1_hbm, Wg2_hbm, bg2_hbm)
    vmem_ws = (flat_ref, We1_ref, be1_ref, We2_ref, be2_ref, We3_ref,
               be3_ref, Wd1_ref, bd1_ref, Wd2_ref, bd2_ref, Wd3_ref,
               bd3_ref, Wg1_ref, bg1_ref, Wg2_ref, bg2_ref)

    def in_copy(k):
        return pltpu.make_async_copy(
            x_hbm.at[pl.ds(k * _CH, _CH), :], x_vmem.at[k], in_sem.at[k])

    def out_copy(k):
        return pltpu.make_async_copy(
            o_vmem.at[k], out_hbm.at[pl.ds(k * _CH, _CH), :], out_sem.at[k])

    def w_copy(j):
        return pltpu.make_async_copy(hbm_ws[j], vmem_ws[j], w_sem.at[j])

    for k in range(_N):
        in_copy(k).start()
    for j in range(len(hbm_ws)):
        w_copy(j).start()
    for j in range(len(hbm_ws)):
        w_copy(j).wait()

    flat = flat_ref[...]
    w2 = jnp.sum(flat * flat, axis=1, keepdims=True)
    flatm2 = -2.0 * flat
    row = jax.lax.broadcasted_iota(jnp.int32, (_G2, 1), 0)
    A = jnp.concatenate([We1_ref[...].T, Wd3_ref[...]],
                        axis=0).astype(jnp.bfloat16)
    Wd3 = Wd3_ref[...]
    bd3row = bd3_ref[...]
    be1c = be1_ref[...].T
    be2c, be3c = be2_ref[...].T, be3_ref[...].T
    bd1c, bd2c = bd1_ref[...].T, bd2_ref[...].T
    bd3c = bd3row.T
    bg1c, bg2c = bg1_ref[...].T, bg2_ref[...].T
    We2, We3 = We2_ref[...], We3_ref[...]
    Wd1, Wd2 = Wd1_ref[...], Wd2_ref[...]
    Wg1, Wg2 = Wg1_ref[...], Wg2_ref[...]
    ones_row = jnp.ones((1, _D), dtype=jnp.bfloat16)

    def compute(x):
        s = w2 + _nt(flatm2, x)
        smin = jnp.min(s, axis=0, keepdims=True)
        idx = jnp.min(jnp.where(s <= smin, row, _G2), axis=0, keepdims=True)
        wi = (idx // _GRID).astype(jnp.float32) * 0.1
        wj = (idx % _GRID).astype(jnp.float32) * 0.1

        xb = x.astype(jnp.bfloat16)
        x2 = jax.lax.dot_general(ones_row, xb * xb, (((1,), (1,)), ((), ())),
                                 preferred_element_type=jnp.float32)
        x_norm = jnp.sqrt(x2)

        P = jax.lax.dot_general(A, xb, (((1,), (1,)), ((), ())),
                                preferred_element_type=jnp.float32)
        h = jnp.tanh(P[0:64] + be1c)
        C = P[64:128]

        h = jnp.tanh(_tt_bf(We2, h) + be2c)
        z_c = _tt_bf(We3, h) + be3c
        h = jnp.tanh(_tt_bf(Wd1, z_c) + bd1c)
        h = jnp.tanh(_tt_bf(Wd2, h) + bd2c)
        x_hat = _tt_bf(Wd3, h) + bd3c

        xxh = jnp.sum(h * C, axis=0, keepdims=True) + _nt(bd3row, x)
        xh2 = jnp.sum(x_hat * x_hat, axis=0, keepdims=True)
        diff2 = jnp.maximum(x2 - 2.0 * xxh + xh2, 0.0)
        rec_e = jnp.sqrt(diff2) / (x_norm + eps)
        rec_c = xxh / (x_norm * jnp.sqrt(xh2) + eps)

        z = jnp.concatenate([z_c, rec_e, rec_c, wi, wj], axis=0)
        g = jnp.tanh(_tt(Wg1, z) + bg1c)
        logits = _tt(Wg2, g) + bg2c
        m = jnp.max(logits, axis=0, keepdims=True)
        e = jnp.exp(logits - m)
        gamma = e / jnp.sum(e, axis=0, keepdims=True)
        return gamma.T

    for k in range(_N):
        in_copy(k).wait()
        o_vmem[k] = compute(x_vmem[k])
        out_copy(k).start()

    for k in range(_N):
        out_copy(k).wait()


def kernel(input, som_weights, We1, be1, We2, be2, We3, be3,
           Wd1, bd1, Wd2, bd2, Wd3, bd3, Wg1, bg1, Wg2, bg2):
    flat = som_weights.reshape(_G2, _D)

    weights = (flat,
               We1, be1.reshape(1, -1), We2, be2.reshape(1, -1),
               We3, be3.reshape(1, -1),
               Wd1, bd1.reshape(1, -1), Wd2, bd2.reshape(1, -1),
               Wd3, bd3.reshape(1, -1),
               Wg1, bg1.reshape(1, -1), Wg2, bg2.reshape(1, -1))

    gamma = pl.pallas_call(
        _pipeline,
        in_specs=[pl.BlockSpec(memory_space=pl.MemorySpace.ANY)
                  for _ in range(1 + len(weights))],
        out_specs=pl.BlockSpec(memory_space=pl.MemorySpace.ANY),
        out_shape=jax.ShapeDtypeStruct((_B, 4), jnp.float32),
        scratch_shapes=[
            pltpu.VMEM((_N, _CH, _D), jnp.float32),
            pltpu.VMEM((_N, _CH, 4), jnp.float32),
        ] + [pltpu.VMEM(w.shape, jnp.float32) for w in weights] + [
            pltpu.SemaphoreType.DMA((_N,)),
            pltpu.SemaphoreType.DMA((_N,)),
            pltpu.SemaphoreType.DMA((len(weights),)),
        ],
    )(input, *weights)
    return gamma


# speedup vs baseline: 1.0916x; 1.0916x over previous
import jax
import jax.numpy as jnp
from jax.experimental import pallas as pl

_GRID = 10
_G2 = _GRID * _GRID
_D = 128
_BB = 8192


def _nt(a, b):
    return jax.lax.dot_general(a, b, (((1,), (1,)), ((), ())),
                               preferred_element_type=jnp.float32)


def _tt(w, act):
    return jax.lax.dot_general(w, act, (((0,), (0,)), ((), ())),
                               preferred_element_type=jnp.float32)


def _tt_bf(w, act):
    return jax.lax.dot_general(w.astype(jnp.bfloat16), act.astype(jnp.bfloat16),
                               (((0,), (0,)), ((), ())),
                               preferred_element_type=jnp.float32)


def _fused(x_ref, flat_ref,
           We1_ref, be1_ref, We2_ref, be2_ref, We3_ref, be3_ref,
           Wd1_ref, bd1_ref, Wd2_ref, bd2_ref, Wd3_ref, bd3_ref,
           Wg1_ref, bg1_ref, Wg2_ref, bg2_ref,
           out_ref):
    eps = 1e-12
    x = x_ref[...]
    flat = flat_ref[...]

    w2 = jnp.sum(flat * flat, axis=1, keepdims=True)
    s = w2 + _nt(-2.0 * flat, x)
    smin = jnp.min(s, axis=0, keepdims=True)
    row = jax.lax.broadcasted_iota(jnp.int32, (_G2, 1), 0)
    idx = jnp.min(jnp.where(s <= smin, row, _G2), axis=0, keepdims=True)
    wi = (idx // _GRID).astype(jnp.float32) * 0.1
    wj = (idx % _GRID).astype(jnp.float32) * 0.1

    ones_row = jnp.ones((1, _D), dtype=jnp.bfloat16)
    xb = x.astype(jnp.bfloat16)
    x2 = jax.lax.dot_general(ones_row, xb * xb, (((1,), (1,)), ((), ())),
                             preferred_element_type=jnp.float32)
    x_norm = jnp.sqrt(x2)

    A = jnp.concatenate([We1_ref[...].T, Wd3_ref[...]], axis=0)
    P = jax.lax.dot_general(A.astype(jnp.bfloat16),
                            x.astype(jnp.bfloat16),
                            (((1,), (1,)), ((), ())),
                            preferred_element_type=jnp.float32)
    h = jnp.tanh(P[0:64] + be1_ref[...].T)
    C = P[64:128]

    h = jnp.tanh(_tt_bf(We2_ref[...], h) + be2_ref[...].T)
    z_c = _tt_bf(We3_ref[...], h) + be3_ref[...].T
    h = jnp.tanh(_tt_bf(Wd1_ref[...], z_c) + bd1_ref[...].T)
    h = jnp.tanh(_tt_bf(Wd2_ref[...], h) + bd2_ref[...].T)
    x_hat = _tt_bf(Wd3_ref[...], h) + bd3_ref[...].T

    xxh = jnp.sum(h * C, axis=0, keepdims=True) + _nt(bd3_ref[...], x)
    xh2 = jnp.sum(x_hat * x_hat, axis=0, keepdims=True)
    diff2 = jnp.maximum(x2 - 2.0 * xxh + xh2, 0.0)
    rec_e = jnp.sqrt(diff2) / (x_norm + eps)
    rec_c = xxh / (x_norm * jnp.sqrt(xh2) + eps)

    z = jnp.concatenate([z_c, rec_e, rec_c, wi, wj], axis=0)
    g = jnp.tanh(_tt(Wg1_ref[...], z) + bg1_ref[...].T)
    logits = _tt(Wg2_ref[...], g) + bg2_ref[...].T
    m = jnp.max(logits, axis=0, keepdims=True)
    e = jnp.exp(logits - m)
    gamma = e / jnp.sum(e, axis=0, keepdims=True)
    out_ref[...] = gamma.T


def kernel(input, som_weights, We1, be1, We2, be2, We3, be3,
           Wd1, bd1, Wd2, bd2, Wd3, bd3, Wg1, bg1, Wg2, bg2):
    B = input.shape[0]
    flat = som_weights.reshape(_G2, _D)

    def full_spec(a):
        nd = a.ndim
        return pl.BlockSpec(a.shape, lambda i: (0,) * nd)

    weights = (flat,
               We1, be1.reshape(1, -1), We2, be2.reshape(1, -1),
               We3, be3.reshape(1, -1),
               Wd1, bd1.reshape(1, -1), Wd2, bd2.reshape(1, -1),
               Wd3, bd3.reshape(1, -1),
               Wg1, bg1.reshape(1, -1), Wg2, bg2.reshape(1, -1))

    gamma = pl.pallas_call(
        _fused,
        grid=(B // _BB,),
        in_specs=[pl.BlockSpec((_BB, _D), lambda i: (i, 0))]
                 + [full_spec(w) for w in weights],
        out_specs=pl.BlockSpec((_BB, 4), lambda i: (i, 0)),
        out_shape=jax.ShapeDtypeStruct((B, 4), jnp.float32),
    )(input, *weights)
    return gamma
